# Initial kernel scaffold; baseline (speedup 1.0000x reference)
import jax
import jax.numpy as jnp
from jax.experimental import pallas as pl


def kernel(x, emb, W1, b1, W2, b2):
    raise NotImplementedError("write your pallas kernel here")


# speedup vs baseline: 2.0769x; 2.0769x over previous
import functools

import jax
import jax.numpy as jnp
from jax import lax
from jax.experimental import pallas as pl
from jax.experimental.pallas import tpu as pltpu
from jax.experimental.pallas import tpu_sc as plsc

VOCAB = 1000000
EMB = 64
HID = 128
BATCH = 16384
SEQ = 200

_NC, _NS = 2, 16
_NW = _NC * _NS
_BPW = BATCH // _NW
_G = 64
_NG = _BPW // _G
_S0 = 128
_S1 = SEQ - _S0


def _pool_body(x_hbm, emb_hbm, out_hbm, idx_v, rows_v, pooled_v, sem0, sem1):
    wid = lax.axis_index("s") * _NC + lax.axis_index("c")
    base = wid * _BPW

    def group(gi, carry):
        g0 = base + gi * _G
        pltpu.sync_copy(x_hbm.at[pl.ds(g0, _G), :], idx_v)

        def elem(e, c):
            cp0 = pltpu.make_async_copy(
                emb_hbm.at[idx_v.at[e, pl.ds(0, _S0)]],
                rows_v.at[pl.ds(0, _S0), :], sem0)
            cp1 = pltpu.make_async_copy(
                emb_hbm.at[idx_v.at[e, pl.ds(_S0, _S1)]],
                rows_v.at[pl.ds(_S0, _S1), :], sem1)
            cp0.start()
            cp1.start()
            cp0.wait()
            cp1.wait()

            def accum(j, acc):
                return tuple(acc[k] + rows_v[j, pl.ds(16 * k, 16)]
                             for k in range(EMB // 16))

            acc = lax.fori_loop(
                0, SEQ, accum,
                tuple(jnp.zeros((16,), jnp.float32) for _ in range(EMB // 16)))
            inv = jnp.float32(1.0 / SEQ)
            for k in range(EMB // 16):
                pooled_v[e, pl.ds(16 * k, 16)] = acc[k] * inv
            return c

        lax.fori_loop(0, _G, elem, 0)
        pltpu.sync_copy(pooled_v, out_hbm.at[pl.ds(g0, _G), :])
        return carry

    lax.fori_loop(0, _NG, group, 0)


_pool = functools.partial(
    pl.kernel,
    mesh=plsc.VectorSubcoreMesh(core_axis_name="c", subcore_axis_name="s"),
    out_type=jax.ShapeDtypeStruct((BATCH, EMB), jnp.float32),
    scratch_types=[
        pltpu.VMEM((_G, SEQ), jnp.int32),
        pltpu.VMEM((SEQ, EMB), jnp.float32),
        pltpu.VMEM((_G, EMB), jnp.float32),
        pltpu.SemaphoreType.DMA,
        pltpu.SemaphoreType.DMA,
    ],
    compiler_params=pltpu.CompilerParams(use_tc_tiling_on_sc=False),
)(_pool_body)


_BBLK = 2048


def _mlp_body(p_ref, w1_ref, b1_ref, w2_ref, b2_ref, o_ref):
    h = jnp.dot(p_ref[...], w1_ref[...], preferred_element_type=jnp.float32)
    h = jnp.maximum(h + b1_ref[...], 0.0)
    z = jnp.dot(h, w2_ref[...], preferred_element_type=jnp.float32) + b2_ref[...]
    o_ref[...] = 1.0 / (1.0 + jnp.exp(-z))


def _mlp(pooled, W1, b1, W2, b2):
    return pl.pallas_call(
        _mlp_body,
        grid=(BATCH // _BBLK,),
        in_specs=[
            pl.BlockSpec((_BBLK, EMB), lambda i: (i, 0)),
            pl.BlockSpec((EMB, HID), lambda i: (0, 0)),
            pl.BlockSpec((1, HID), lambda i: (0, 0)),
            pl.BlockSpec((HID, 1), lambda i: (0, 0)),
            pl.BlockSpec((1, 1), lambda i: (0, 0)),
        ],
        out_specs=pl.BlockSpec((_BBLK, 1), lambda i: (i, 0)),
        out_shape=jax.ShapeDtypeStruct((BATCH, 1), jnp.float32),
    )(pooled, W1, b1.reshape(1, HID), W2, b2.reshape(1, 1))


def kernel(x, emb, W1, b1, W2, b2):
    pooled = _pool(x.astype(jnp.int32), emb)
    return _mlp(pooled, W1, b1, W2, b2)


# speedup vs baseline: 2.9220x; 1.4069x over previous
import functools

import jax
import jax.numpy as jnp
from jax import lax
from jax.experimental import pallas as pl
from jax.experimental.pallas import tpu as pltpu
from jax.experimental.pallas import tpu_sc as plsc

VOCAB = 1000000
EMB = 64
HID = 128
BATCH = 16384
SEQ = 200

_NC, _NS = 2, 16
_NW = _NC * _NS
_BPW = BATCH // _NW
_G = 64
_NG = _BPW // _G
_S0 = 128
_S1 = SEQ - _S0


def _pool_body(x_hbm, emb_hbm, out_hbm, idx_v, rows_a, rows_b, pooled_v,
               sem_a, sem_b):
    wid = lax.axis_index("s") * _NC + lax.axis_index("c")
    base = wid * _BPW

    def copies(e, rows_ref, sem):
        return (
            pltpu.make_async_copy(
                emb_hbm.at[idx_v.at[e, pl.ds(0, _S0)]],
                rows_ref.at[pl.ds(0, _S0), :], sem),
            pltpu.make_async_copy(
                emb_hbm.at[idx_v.at[e, pl.ds(_S0, _S1)]],
                rows_ref.at[pl.ds(_S0, _S1), :], sem),
        )

    def start(e, rows_ref, sem):
        for cp in copies(e, rows_ref, sem):
            cp.start()

    def wait(e, rows_ref, sem):
        for cp in copies(e, rows_ref, sem):
            cp.wait()

    def accum_into(rows_ref, e):
        def body(j, acc):
            return tuple(acc[k] + rows_ref[j, pl.ds(16 * k, 16)]
                         for k in range(EMB // 16))

        acc = lax.fori_loop(
            0, SEQ, body,
            tuple(jnp.zeros((16,), jnp.float32) for _ in range(EMB // 16)),
            unroll=4)
        inv = jnp.float32(1.0 / SEQ)
        for k in range(EMB // 16):
            pooled_v[e, pl.ds(16 * k, 16)] = acc[k] * inv

    def group(gi, carry):
        g0 = base + gi * _G
        pltpu.sync_copy(x_hbm.at[pl.ds(g0, _G), :], idx_v)
        start(0, rows_a, sem_a)

        def pair(p, c):
            e = 2 * p
            start(e + 1, rows_b, sem_b)
            wait(e, rows_a, sem_a)
            accum_into(rows_a, e)

            @pl.when(p + 1 < _G // 2)
            def _():
                start(e + 2, rows_a, sem_a)

            wait(e + 1, rows_b, sem_b)
            accum_into(rows_b, e + 1)
            return c

        lax.fori_loop(0, _G // 2, pair, 0)
        pltpu.sync_copy(pooled_v, out_hbm.at[pl.ds(g0, _G), :])
        return carry

    lax.fori_loop(0, _NG, group, 0)


_pool = functools.partial(
    pl.kernel,
    mesh=plsc.VectorSubcoreMesh(core_axis_name="c", subcore_axis_name="s"),
    out_type=jax.ShapeDtypeStruct((BATCH, EMB), jnp.float32),
    scratch_types=[
        pltpu.VMEM((_G, SEQ), jnp.int32),
        pltpu.VMEM((SEQ, EMB), jnp.float32),
        pltpu.VMEM((SEQ, EMB), jnp.float32),
        pltpu.VMEM((_G, EMB), jnp.float32),
        pltpu.SemaphoreType.DMA,
        pltpu.SemaphoreType.DMA,
    ],
    compiler_params=pltpu.CompilerParams(use_tc_tiling_on_sc=False),
)(_pool_body)


_BBLK = 2048


def _mlp_body(p_ref, w1_ref, b1_ref, w2_ref, b2_ref, o_ref):
    h = jnp.dot(p_ref[...], w1_ref[...], preferred_element_type=jnp.float32)
    h = jnp.maximum(h + b1_ref[...], 0.0)
    z = jnp.dot(h, w2_ref[...], preferred_element_type=jnp.float32) + b2_ref[...]
    o_ref[...] = 1.0 / (1.0 + jnp.exp(-z))


def _mlp(pooled, W1, b1, W2, b2):
    return pl.pallas_call(
        _mlp_body,
        grid=(BATCH // _BBLK,),
        in_specs=[
            pl.BlockSpec((_BBLK, EMB), lambda i: (i, 0)),
            pl.BlockSpec((EMB, HID), lambda i: (0, 0)),
            pl.BlockSpec((1, HID), lambda i: (0, 0)),
            pl.BlockSpec((HID, 1), lambda i: (0, 0)),
            pl.BlockSpec((1, 1), lambda i: (0, 0)),
        ],
        out_specs=pl.BlockSpec((_BBLK, 1), lambda i: (i, 0)),
        out_shape=jax.ShapeDtypeStruct((BATCH, 1), jnp.float32),
    )(pooled, W1, b1.reshape(1, HID), W2, b2.reshape(1, 1))


def kernel(x, emb, W1, b1, W2, b2):
    pooled = _pool(x.astype(jnp.int32), emb)
    return _mlp(pooled, W1, b1, W2, b2)
